# baseline (speedup 1.0000x reference)
import jax
import jax.numpy as jnp
from jax import lax
from jax.experimental import pallas as pl
from jax.experimental.pallas import tpu as pltpu, tpu_sc as plsc

V = 1000000
D = 64
B = 16384
NC = 2
NS = 16
NW = NC * NS
SW = 256
SSH = 8
VFULL = (V // SW) * SW
NSLAB = VFULL // SW
TAILS = NSLAB
LCAP = B + 32
OUTROWS = B + 128
NB = 8
NBUK = 8
TPB = 16
XCH = 2048
PSH_T = 22
PSH_C = 14


def _it16():
    return lax.iota(jnp.int32, 16)


def _splat(x):
    return jnp.full((16,), x, dtype=jnp.int32)


def _body(wt_hbm, tail_hbm, idx_hbm, out_hbm,
          xbuf_v, la_v, lb_v, slab_v, stage_v, pstage_v, tail_v,
          sem_x, sem_s, sem_o):
    wid = lax.axis_index("s") * NC + lax.axis_index("c")
    sentinel = B + wid
    it16 = _it16()
    ntrips = (NSLAB - wid + NW - 1) // NW

    def slab_dma(g, slot):
        colbase = (wid + g * NW) * SW
        return pltpu.make_async_copy(
            wt_hbm.at[:, pl.ds(colbase, SW)], slab_v.at[slot], sem_s)

    for _g in range(4):
        slab_dma(_g, _g).start()
    pltpu.sync_copy(tail_hbm, tail_v)

    def x_dma(c, slot):
        return pltpu.make_async_copy(
            idx_hbm.at[pl.ds(c * XCH, XCH)], xbuf_v.at[slot], sem_x)

    x_dma(0, 0).start()

    def bin_chunk(c, cnt):
        slot = c % 2
        x_dma(c, slot).wait()

        @pl.when(c + 1 < B // XCH)
        def _():
            x_dma(c + 1, (c + 1) % 2).start()

        def bin16(j, cnt):
            v = xbuf_v[slot, pl.ds(j * 16, 16)]
            s = v >> SSH
            m = (s & (NW - 1)) == wid
            mi = m.astype(jnp.int32)
            offs = cnt + jnp.cumsum(mi) - mi
            pos = c * XCH + j * 16 + it16
            g = (s - wid) >> 5
            e = (g << PSH_T) | ((v & (SW - 1)) << PSH_C) | pos
            plsc.store_scatter(la_v, [offs], e, mask=m)
            return cnt + jnp.sum(mi)

        return lax.fori_loop(0, XCH // 16, bin16, cnt)

    cnt = lax.fori_loop(0, B // XCH, bin_chunk, 0)
    nchunks = (cnt + 15) // 16

    def count16(j, cs):
        base = j * 16
        e = la_v[pl.ds(base, 16)]
        valid = (base + it16) < cnt
        bk = e >> (PSH_T + 4)
        return tuple(cs[b] + jnp.sum(((bk == b) & valid).astype(jnp.int32))
                     for b in range(NBUK))

    counts = lax.fori_loop(0, nchunks, count16, (0,) * NBUK)
    boff = [0]
    for b in range(NBUK):
        boff.append(boff[-1] + counts[b])

    def place16(j, offs):
        base = j * 16
        e = la_v[pl.ds(base, 16)]
        valid = (base + it16) < cnt
        bk = e >> (PSH_T + 4)
        new = []
        for b in range(NBUK):
            m = (bk == b) & valid
            mi = m.astype(jnp.int32)
            dst = offs[b] + jnp.cumsum(mi) - mi
            plsc.store_scatter(lb_v, [dst], e, mask=m)
            new.append(offs[b] + jnp.sum(mi))
        return tuple(new)

    lax.fori_loop(0, nchunks, place16, tuple(boff[:NBUK]))

    def filter_slab(g_cur, lo, hi):
        def filt(j, hc):
            base = lo + j * 16
            e = lb_v[pl.ds(base, 16)]
            m = ((base + it16) < hi) & ((e >> PSH_T) == g_cur)
            mi = m.astype(jnp.int32)
            dst = hc + jnp.cumsum(mi) - mi
            plsc.store_scatter(la_v, [dst], e, mask=m)
            return hc + jnp.sum(mi)

        return lax.fori_loop(0, (hi - lo + 15) // 16, filt, 0)

    def out_dma(bank):
        return pltpu.make_async_copy(
            stage_v.at[bank], out_hbm.at[pstage_v.at[bank]], sem_o)

    def gather_groups(buf, colmask, hc, gctr):
        def group(k, gctr):
            base = k * 16
            gvalid = (base + it16) < hc
            e = la_v[pl.ds(base, 16)]
            colv = (e >> PSH_C) & colmask
            posv = e & (B - 1)
            bank = gctr % NB

            @pl.when(gctr >= NB)
            def _():
                out_dma(bank).wait()

            bsp = _splat(bank)
            for d in range(D):
                vals = plsc.load_gather(buf, [_splat(d), colv], mask=gvalid)
                plsc.store_scatter(stage_v, [bsp, it16, _splat(d)], vals,
                                   mask=gvalid)
            pf = jnp.where(gvalid, posv, _splat(sentinel))
            plsc.store_scatter(pstage_v, [bsp, it16], pf)
            out_dma(bank).start()
            return gctr + 1

        return lax.fori_loop(0, (hc + 15) // 16, group, gctr)

    def do_bucket(b, gctr, lo, hi):
        def do_slab(t, gctr):
            g = b * TPB + t
            slot = g % 4
            hc = filter_slab(g, lo, hi)
            slab_dma(g, slot).wait()
            gctr = gather_groups(slab_v.at[slot], SW - 1, hc, gctr)

            @pl.when(g + 4 < ntrips)
            def _():
                slab_dma(g + 4, slot).start()

            return gctr

        trips = jnp.clip(ntrips - b * TPB, 0, TPB)
        return lax.fori_loop(0, trips, do_slab, gctr)

    gctr = 0
    for b in range(NBUK):
        gctr = do_bucket(b, gctr, boff[b], boff[b + 1])

    g_tail = (TAILS - wid) >> 5
    hc = filter_slab(g_tail, boff[NBUK - 1], boff[NBUK])
    hc = jnp.where(wid == TAILS % NW, hc, 0)
    gctr = gather_groups(tail_v, 127, hc, gctr)

    def drain(i, _):
        out_dma(i % NB).wait()
        return 0

    lax.fori_loop(0, jnp.minimum(gctr, NB), drain, 0)


def kernel(x, weight):
    wt = weight.T
    tail = jnp.concatenate(
        [wt[:, VFULL:], jnp.zeros((D, 128 - (V - VFULL)), jnp.float32)],
        axis=1)
    f = pl.kernel(
        _body,
        out_type=jax.ShapeDtypeStruct((OUTROWS, 128), jnp.float32),
        mesh=plsc.VectorSubcoreMesh(core_axis_name="c", subcore_axis_name="s"),
        scratch_types=[
            pltpu.VMEM((2, XCH), jnp.int32),
            pltpu.VMEM((LCAP,), jnp.int32),
            pltpu.VMEM((LCAP,), jnp.int32),
            pltpu.VMEM((4, D, SW), jnp.float32),
            pltpu.VMEM((NB, 16, 128), jnp.float32),
            pltpu.VMEM((NB, 16), jnp.int32),
            pltpu.VMEM((D, 128), jnp.float32),
            pltpu.SemaphoreType.DMA,
            pltpu.SemaphoreType.DMA,
            pltpu.SemaphoreType.DMA,
        ],
        compiler_params=pltpu.CompilerParams(
            use_tc_tiling_on_sc=True, needs_layout_passes=False),
    )
    outp = f(wt, tail, x.astype(jnp.int32))
    return outp[:B, :D]


# speedup vs baseline: 1.4542x; 1.4542x over previous
import jax
import jax.numpy as jnp
from jax import lax
from jax.experimental import pallas as pl
from jax.experimental.pallas import tpu as pltpu, tpu_sc as plsc

V = 1000000
D = 64
B = 16384
NC = 2
NS = 16
NW = NC * NS
SW = 512
SSH = 9
VFULL = (V // SW) * SW
NSLAB = VFULL // SW
TAILS = NSLAB
LCAP = B + 32
OUTROWS = B + 128
NB = 8
NBUK = 8
TPB = 8
XCH = 2048
PSH_T = 23
PSH_C = 14


def _it16():
    return lax.iota(jnp.int32, 16)


def _splat(x):
    return jnp.full((16,), x, dtype=jnp.int32)


def _body(wt_hbm, tail_hbm, idx_hbm, out_hbm,
          xbuf_v, la_v, lb_v, slab_v, stage_v, pstage_v, tail_v,
          sem_x, sem_s, sem_o):
    wid = lax.axis_index("s") * NC + lax.axis_index("c")
    sentinel = B + wid
    it16 = _it16()
    ntrips = (NSLAB - wid + NW - 1) // NW

    def slab_dma(g, slot):
        colbase = (wid + g * NW) * SW
        return pltpu.make_async_copy(
            wt_hbm.at[:, pl.ds(colbase, SW)], slab_v.at[slot], sem_s)

    slab_dma(0, 0).start()
    slab_dma(1, 1).start()
    pltpu.sync_copy(tail_hbm, tail_v)

    def x_dma(c, slot):
        return pltpu.make_async_copy(
            idx_hbm.at[pl.ds(c * XCH, XCH)], xbuf_v.at[slot], sem_x)

    x_dma(0, 0).start()

    def bin_chunk(c, cnt):
        slot = c % 2
        x_dma(c, slot).wait()

        @pl.when(c + 1 < B // XCH)
        def _():
            x_dma(c + 1, (c + 1) % 2).start()

        def bin16(j, cnt):
            v = xbuf_v[slot, pl.ds(j * 16, 16)]
            s = v >> SSH
            m = (s & (NW - 1)) == wid
            mi = m.astype(jnp.int32)
            offs = cnt + jnp.cumsum(mi) - mi
            pos = c * XCH + j * 16 + it16
            g = (s - wid) >> 5
            e = (g << PSH_T) | ((v & (SW - 1)) << PSH_C) | pos
            plsc.store_scatter(la_v, [offs], e, mask=m)
            return cnt + jnp.sum(mi)

        return lax.fori_loop(0, XCH // 16, bin16, cnt)

    cnt = lax.fori_loop(0, B // XCH, bin_chunk, 0)
    nchunks = (cnt + 15) // 16

    def count16(j, cs):
        base = j * 16
        e = la_v[pl.ds(base, 16)]
        valid = (base + it16) < cnt
        bk = e >> (PSH_T + 3)
        return tuple(cs[b] + jnp.sum(((bk == b) & valid).astype(jnp.int32))
                     for b in range(NBUK))

    counts = lax.fori_loop(0, nchunks, count16, (0,) * NBUK)
    boff = [0]
    for b in range(NBUK):
        boff.append(boff[-1] + counts[b])

    def place16(j, offs):
        base = j * 16
        e = la_v[pl.ds(base, 16)]
        valid = (base + it16) < cnt
        bk = e >> (PSH_T + 3)
        new = []
        for b in range(NBUK):
            m = (bk == b) & valid
            mi = m.astype(jnp.int32)
            dst = offs[b] + jnp.cumsum(mi) - mi
            plsc.store_scatter(lb_v, [dst], e, mask=m)
            new.append(offs[b] + jnp.sum(mi))
        return tuple(new)

    lax.fori_loop(0, nchunks, place16, tuple(boff[:NBUK]))

    def filter_slab(g_cur, lo, hi):
        def filt(j, hc):
            base = lo + j * 16
            e = lb_v[pl.ds(base, 16)]
            m = ((base + it16) < hi) & ((e >> PSH_T) == g_cur)
            mi = m.astype(jnp.int32)
            dst = hc + jnp.cumsum(mi) - mi
            plsc.store_scatter(la_v, [dst], e, mask=m)
            return hc + jnp.sum(mi)

        return lax.fori_loop(0, (hi - lo + 15) // 16, filt, 0)

    def out_dma(bank):
        return pltpu.make_async_copy(
            stage_v.at[bank], out_hbm.at[pstage_v.at[bank]], sem_o)

    def gather_groups(buf, colmask, hc, gctr):
        def group(k, gctr):
            base = k * 16
            gvalid = (base + it16) < hc
            e = la_v[pl.ds(base, 16)]
            colv = (e >> PSH_C) & colmask
            posv = e & (B - 1)
            bank = gctr % NB

            @pl.when(gctr >= NB)
            def _():
                out_dma(bank).wait()

            bsp = _splat(bank)
            for h in range(16):
                csp = _splat(colv[h])
                for q in range(D // 16):
                    dvec = it16 + q * 16
                    vals = plsc.load_gather(buf, [dvec, csp])
                    plsc.store_scatter(stage_v, [bsp, _splat(h), dvec], vals)
            pf = jnp.where(gvalid, posv, _splat(sentinel))
            plsc.store_scatter(pstage_v, [bsp, it16], pf)
            out_dma(bank).start()
            return gctr + 1

        return lax.fori_loop(0, (hc + 15) // 16, group, gctr)

    def do_bucket(b, gctr, lo, hi):
        def do_slab(t, gctr):
            g = b * TPB + t
            slot = g % 2
            hc = filter_slab(g, lo, hi)
            slab_dma(g, slot).wait()
            gctr = gather_groups(slab_v.at[slot], SW - 1, hc, gctr)

            @pl.when(g + 2 < ntrips)
            def _():
                slab_dma(g + 2, slot).start()

            return gctr

        trips = jnp.clip(ntrips - b * TPB, 0, TPB)
        return lax.fori_loop(0, trips, do_slab, gctr)

    gctr = 0
    for b in range(NBUK):
        gctr = do_bucket(b, gctr, boff[b], boff[b + 1])

    g_tail = (TAILS - wid) >> 5
    hc = filter_slab(g_tail, boff[NBUK - 1], boff[NBUK])
    hc = jnp.where(wid == TAILS % NW, hc, 0)
    gctr = gather_groups(tail_v, 127, hc, gctr)

    def drain(i, _):
        out_dma(i % NB).wait()
        return 0

    lax.fori_loop(0, jnp.minimum(gctr, NB), drain, 0)


def kernel(x, weight):
    wt = weight.T
    tail = jnp.concatenate(
        [wt[:, VFULL:], jnp.zeros((D, 128 - (V - VFULL)), jnp.float32)],
        axis=1)
    f = pl.kernel(
        _body,
        out_type=jax.ShapeDtypeStruct((OUTROWS, 128), jnp.float32),
        mesh=plsc.VectorSubcoreMesh(core_axis_name="c", subcore_axis_name="s"),
        scratch_types=[
            pltpu.VMEM((2, XCH), jnp.int32),
            pltpu.VMEM((LCAP,), jnp.int32),
            pltpu.VMEM((LCAP,), jnp.int32),
            pltpu.VMEM((2, D, SW), jnp.float32),
            pltpu.VMEM((NB, 16, 128), jnp.float32),
            pltpu.VMEM((NB, 16), jnp.int32),
            pltpu.VMEM((D, 128), jnp.float32),
            pltpu.SemaphoreType.DMA,
            pltpu.SemaphoreType.DMA,
            pltpu.SemaphoreType.DMA,
        ],
        compiler_params=pltpu.CompilerParams(
            use_tc_tiling_on_sc=True, needs_layout_passes=False),
    )
    outp = f(wt, tail, x.astype(jnp.int32))
    return outp[:B, :D]


# speedup vs baseline: 1.4657x; 1.0079x over previous
import jax
import jax.numpy as jnp
from jax import lax
from jax.experimental import pallas as pl
from jax.experimental.pallas import tpu as pltpu, tpu_sc as plsc

V = 1000000
D = 64
B = 16384
NC = 2
NS = 16
NW = NC * NS
SW = 512
SSH = 9
VFULL = (V // SW) * SW
NSLAB = VFULL // SW
TAILS = NSLAB
LCAP = B + 32
OUTROWS = B + 128
NB = 8
NBUK = 8
TPB = 8
XCH = 2048
PSH_T = 23
PSH_C = 14


def _it16():
    return lax.iota(jnp.int32, 16)


def _splat(x):
    return jnp.full((16,), x, dtype=jnp.int32)


def _body(wt_hbm, tail_hbm, idx_hbm, out_hbm,
          xbuf_v, la_v, lb_v, slab_v, stage_v, pstage_v, tail_v,
          sem_x, sem_s, sem_o):
    wid = lax.axis_index("s") * NC + lax.axis_index("c")
    sentinel = B + wid
    it16 = _it16()
    ntrips = (NSLAB - wid + NW - 1) // NW

    def slab_dma(g, slot):
        colbase = (wid + g * NW) * SW
        return pltpu.make_async_copy(
            wt_hbm.at[:, pl.ds(colbase, SW)], slab_v.at[slot], sem_s)

    slab_dma(0, 0).start()
    slab_dma(1, 1).start()
    pltpu.sync_copy(tail_hbm, tail_v)

    def x_dma(c, slot):
        return pltpu.make_async_copy(
            idx_hbm.at[pl.ds(c * XCH, XCH)], xbuf_v.at[slot], sem_x)

    x_dma(0, 0).start()

    def bin_chunk(c, cnt):
        slot = c % 2
        x_dma(c, slot).wait()

        @pl.when(c + 1 < B // XCH)
        def _():
            x_dma(c + 1, (c + 1) % 2).start()

        def bin16(j, cnt):
            v = xbuf_v[slot, pl.ds(j * 16, 16)]
            s = v >> SSH
            m = (s & (NW - 1)) == wid
            mi = m.astype(jnp.int32)
            offs = cnt + jnp.cumsum(mi) - mi
            pos = c * XCH + j * 16 + it16
            g = (s - wid) >> 5
            e = (g << PSH_T) | ((v & (SW - 1)) << PSH_C) | pos
            plsc.store_scatter(la_v, [offs], e, mask=m)
            return cnt + jnp.sum(mi)

        return lax.fori_loop(0, XCH // 16, bin16, cnt)

    cnt = lax.fori_loop(0, B // XCH, bin_chunk, 0)
    nchunks = (cnt + 15) // 16

    def count16(j, cs):
        base = j * 16
        e = la_v[pl.ds(base, 16)]
        valid = (base + it16) < cnt
        bk = e >> (PSH_T + 3)
        return tuple(cs[b] + jnp.sum(((bk == b) & valid).astype(jnp.int32))
                     for b in range(NBUK))

    counts = lax.fori_loop(0, nchunks, count16, (0,) * NBUK)
    boff = [0]
    for b in range(NBUK):
        boff.append(boff[-1] + counts[b])

    def place16(j, offs):
        base = j * 16
        e = la_v[pl.ds(base, 16)]
        valid = (base + it16) < cnt
        bk = e >> (PSH_T + 3)
        new = []
        for b in range(NBUK):
            m = (bk == b) & valid
            mi = m.astype(jnp.int32)
            dst = offs[b] + jnp.cumsum(mi) - mi
            plsc.store_scatter(lb_v, [dst], e, mask=m)
            new.append(offs[b] + jnp.sum(mi))
        return tuple(new)

    lax.fori_loop(0, nchunks, place16, tuple(boff[:NBUK]))

    def filter_slab(g_cur, lo, hi):
        def filt(j, hc):
            base = lo + j * 16
            e = lb_v[pl.ds(base, 16)]
            m = ((base + it16) < hi) & ((e >> PSH_T) == g_cur)
            mi = m.astype(jnp.int32)
            dst = hc + jnp.cumsum(mi) - mi
            plsc.store_scatter(la_v, [dst], e, mask=m)
            return hc + jnp.sum(mi)

        return lax.fori_loop(0, (hi - lo + 15) // 16, filt, 0)

    def out_dma(bank):
        return pltpu.make_async_copy(
            stage_v.at[bank], out_hbm.at[pstage_v.at[bank]], sem_o)

    def gather_groups(buf, colmask, hc, gctr):
        def group(k, gctr):
            base = k * 16
            gvalid = (base + it16) < hc
            e = la_v[pl.ds(base, 16)]
            colv = (e >> PSH_C) & colmask
            posv = e & (B - 1)
            bank = gctr % NB

            @pl.when(gctr >= NB)
            def _():
                out_dma(bank).wait()

            bsp = _splat(bank)
            for d in range(D):
                vals = plsc.load_gather(buf, [_splat(d), colv], mask=gvalid)
                plsc.store_scatter(stage_v, [bsp, it16, _splat(d)], vals,
                                   mask=gvalid)
            pf = jnp.where(gvalid, posv, _splat(sentinel))
            plsc.store_scatter(pstage_v, [bsp, it16], pf)
            out_dma(bank).start()
            return gctr + 1

        return lax.fori_loop(0, (hc + 15) // 16, group, gctr)

    def do_bucket(b, gctr, lo, hi):
        def do_slab(t, gctr):
            g = b * TPB + t
            slot = g % 2
            slab_dma(g, slot).wait()
            hc = filter_slab(g, lo, hi)
            gctr = gather_groups(slab_v.at[slot], SW - 1, hc, gctr)

            @pl.when(g + 2 < ntrips)
            def _():
                slab_dma(g + 2, slot).start()

            return gctr

        trips = jnp.clip(ntrips - b * TPB, 0, TPB)
        return lax.fori_loop(0, trips, do_slab, gctr)

    gctr = 0
    for b in range(NBUK):
        gctr = do_bucket(b, gctr, boff[b], boff[b + 1])

    g_tail = (TAILS - wid) >> 5
    hc = filter_slab(g_tail, boff[NBUK - 1], boff[NBUK])
    hc = jnp.where(wid == TAILS % NW, hc, 0)
    gctr = gather_groups(tail_v, 127, hc, gctr)

    def drain(i, _):
        out_dma(i % NB).wait()
        return 0

    lax.fori_loop(0, jnp.minimum(gctr, NB), drain, 0)


def kernel(x, weight):
    wt = weight.T
    tail = jnp.concatenate(
        [wt[:, VFULL:], jnp.zeros((D, 128 - (V - VFULL)), jnp.float32)],
        axis=1)
    f = pl.kernel(
        _body,
        out_type=jax.ShapeDtypeStruct((OUTROWS, 128), jnp.float32),
        mesh=plsc.VectorSubcoreMesh(core_axis_name="c", subcore_axis_name="s"),
        scratch_types=[
            pltpu.VMEM((2, XCH), jnp.int32),
            pltpu.VMEM((LCAP,), jnp.int32),
            pltpu.VMEM((LCAP,), jnp.int32),
            pltpu.VMEM((2, D, SW), jnp.float32),
            pltpu.VMEM((NB, 16, 128), jnp.float32),
            pltpu.VMEM((NB, 16), jnp.int32),
            pltpu.VMEM((D, 128), jnp.float32),
            pltpu.SemaphoreType.DMA,
            pltpu.SemaphoreType.DMA,
            pltpu.SemaphoreType.DMA,
        ],
        compiler_params=pltpu.CompilerParams(
            use_tc_tiling_on_sc=True, needs_layout_passes=False),
    )
    outp = f(wt, tail, x.astype(jnp.int32))
    return outp[:B, :D]
